# initial kernel scaffold (unmeasured)
import jax
import jax.numpy as jnp
from jax import lax
from jax.experimental import pallas as pl
from jax.experimental.pallas import tpu as pltpu

N_DEV = 16


def kernel(A, B):
    m, k = A.shape
    k2, n = B.shape
    assert k == k2
    chunk_m = m // N_DEV

    def body(a_ref, b_ref, out_ref, acc_ref, rs_buf,
             rs_send_sems, rs_recv_sems, ag_send_sems, ag_recv_sems):
        me = lax.axis_index("i")
        left = lax.rem(me + (N_DEV - 1), N_DEV)
        right = lax.rem(me + 1, N_DEV)

        barrier_sem = pltpu.get_barrier_semaphore()
        for nbr in (left, right):
            pl.semaphore_signal(
                barrier_sem, inc=1,
                device_id=(nbr,), device_id_type=pl.DeviceIdType.MESH,
            )
        pl.semaphore_wait(barrier_sem, 2)

        acc_ref[:, :] = jnp.dot(
            a_ref[:, :].astype(jnp.bfloat16),
            b_ref[:, :].astype(jnp.bfloat16),
            preferred_element_type=jnp.float32,
        )

        for s in range(N_DEV - 1):
            c = lax.rem(me - s + N_DEV, N_DEV)
            if s > 0:
                acc_ref[pl.ds(c * chunk_m, chunk_m), :] = (
                    acc_ref[pl.ds(c * chunk_m, chunk_m), :] + rs_buf[s - 1, :, :]
                )
            rdma = pltpu.make_async_remote_copy(
                src_ref=acc_ref.at[pl.ds(c * chunk_m, chunk_m), :],
                dst_ref=rs_buf.at[s],
                send_sem=rs_send_sems.at[s],
                recv_sem=rs_recv_sems.at[s],
                device_id=(right,),
                device_id_type=pl.DeviceIdType.MESH,
            )
            rdma.start()
            rdma.wait()

        o = lax.rem(me + 1, N_DEV)
        z = acc_ref[pl.ds(o * chunk_m, chunk_m), :] + rs_buf[N_DEV - 2, :, :]
        g = 0.5 * z * (1.0 + jnp.tanh(0.7978845608 * (z + 0.044715 * z * z * z)))
        out_ref[pl.ds(o * chunk_m, chunk_m), :] = g.astype(jnp.bfloat16)

        for s in range(N_DEV - 1):
            gsend = lax.rem(me + 1 - s + N_DEV, N_DEV)
            grecv = lax.rem(me - s + N_DEV, N_DEV)
            rdma = pltpu.make_async_remote_copy(
                src_ref=out_ref.at[pl.ds(gsend * chunk_m, chunk_m), :],
                dst_ref=out_ref.at[pl.ds(gsend * chunk_m, chunk_m), :],
                send_sem=ag_send_sems.at[s],
                recv_sem=ag_recv_sems.at[s],
                device_id=(right,),
                device_id_type=pl.DeviceIdType.MESH,
            )
            rdma.start()
            rdma.wait()
            del grecv

    return pl.pallas_call(
        body,
        out_shape=jax.ShapeDtypeStruct((m, n), jnp.bfloat16),
        in_specs=[
            pl.BlockSpec(memory_space=pltpu.VMEM),
            pl.BlockSpec(memory_space=pltpu.VMEM),
        ],
        out_specs=pl.BlockSpec(memory_space=pltpu.VMEM),
        scratch_shapes=[
            pltpu.VMEM((m, n), jnp.float32),
            pltpu.VMEM((N_DEV - 1, chunk_m, n), jnp.float32),
            pltpu.SemaphoreType.DMA((N_DEV - 1,)),
            pltpu.SemaphoreType.DMA((N_DEV - 1,)),
            pltpu.SemaphoreType.DMA((N_DEV - 1,)),
            pltpu.SemaphoreType.DMA((N_DEV - 1,)),
        ],
        compiler_params=pltpu.CompilerParams(collective_id=0),
    )(A, B)


# baseline (device time: 339100 ns/iter reference)
import jax
import jax.numpy as jnp
from jax import lax
from jax.experimental import pallas as pl
from jax.experimental.pallas import tpu as pltpu

N_DEV = 16


def kernel(A, B):
    m, k = A.shape
    k2, n = B.shape
    assert k == k2
    chunk_m = m // N_DEV

    def body(a_ref, b_ref, out_ref, acc_ref, rs_buf,
             rs_send_sems, rs_recv_sems, ag_send_sems, ag_recv_sems):
        me = lax.axis_index("i")
        left = lax.rem(me + (N_DEV - 1), N_DEV)
        right = lax.rem(me + 1, N_DEV)

        barrier_sem = pltpu.get_barrier_semaphore()
        for nbr in (left, right):
            pl.semaphore_signal(
                barrier_sem, inc=1,
                device_id=(nbr,), device_id_type=pl.DeviceIdType.MESH,
            )
        pl.semaphore_wait(barrier_sem, 2)

        acc_ref[:, :] = jnp.dot(
            a_ref[:, :].astype(jnp.bfloat16),
            b_ref[:, :].astype(jnp.bfloat16),
            preferred_element_type=jnp.float32,
        )

        for s in range(N_DEV - 1):
            c = lax.rem(me - s + N_DEV, N_DEV)
            if s > 0:
                acc_ref[pl.ds(c * chunk_m, chunk_m), :] = (
                    acc_ref[pl.ds(c * chunk_m, chunk_m), :] + rs_buf[s - 1, :, :]
                )
            rdma = pltpu.make_async_remote_copy(
                src_ref=acc_ref.at[pl.ds(c * chunk_m, chunk_m), :],
                dst_ref=rs_buf.at[s],
                send_sem=rs_send_sems.at[s],
                recv_sem=rs_recv_sems.at[s],
                device_id=(right,),
                device_id_type=pl.DeviceIdType.MESH,
            )
            rdma.start()
            rdma.wait()

        o = lax.rem(me + 1, N_DEV)
        z = acc_ref[pl.ds(o * chunk_m, chunk_m), :] + rs_buf[N_DEV - 2, :, :]
        g = 0.5 * z * (1.0 + jnp.tanh(0.7978845608 * (z + 0.044715 * z * z * z)))
        out_ref[pl.ds(o * chunk_m, chunk_m), :] = g.astype(jnp.bfloat16)

        for s in range(N_DEV - 1):
            gsend = lax.rem(me + 1 - s + N_DEV, N_DEV)
            grecv = lax.rem(me - s + N_DEV, N_DEV)
            rdma = pltpu.make_async_remote_copy(
                src_ref=out_ref.at[pl.ds(gsend * chunk_m, chunk_m), :],
                dst_ref=out_ref.at[pl.ds(gsend * chunk_m, chunk_m), :],
                send_sem=ag_send_sems.at[s],
                recv_sem=ag_recv_sems.at[s],
                device_id=(right,),
                device_id_type=pl.DeviceIdType.MESH,
            )
            rdma.start()
            rdma.wait()
            del grecv

    return pl.pallas_call(
        body,
        out_shape=jax.ShapeDtypeStruct((m, n), jnp.bfloat16),
        in_specs=[
            pl.BlockSpec(memory_space=pltpu.VMEM),
            pl.BlockSpec(memory_space=pltpu.VMEM),
        ],
        out_specs=pl.BlockSpec(memory_space=pltpu.VMEM),
        scratch_shapes=[
            pltpu.VMEM((m, n), jnp.float32),
            pltpu.VMEM((N_DEV - 1, chunk_m, n), jnp.float32),
            pltpu.SemaphoreType.DMA((N_DEV - 1,)),
            pltpu.SemaphoreType.DMA((N_DEV - 1,)),
            pltpu.SemaphoreType.DMA((N_DEV - 1,)),
            pltpu.SemaphoreType.DMA((N_DEV - 1,)),
        ],
        compiler_params=pltpu.CompilerParams(
            collective_id=0,
            vmem_limit_bytes=100 * 1024 * 1024,
        ),
    )(A, B)


# device time: 211251 ns/iter; 1.6052x vs baseline; 1.6052x over previous
import jax
import jax.numpy as jnp
from jax import lax
from jax.experimental import pallas as pl
from jax.experimental.pallas import tpu as pltpu

N_DEV = 16


def kernel(A, B):
    m, k = A.shape
    k2, n = B.shape
    assert k == k2
    cm = m // N_DEV
    nh = n // 2

    f32 = jnp.float32
    bf16 = jnp.bfloat16

    def body(a_ref, b_ref, out_ref, acc_ref, bufR, bufL,
             rs_sems, ag_sems):
        me = lax.axis_index("i")
        left = lax.rem(me + (N_DEV - 1), N_DEV)
        right = lax.rem(me + 1, N_DEV)

        barrier_sem = pltpu.get_barrier_semaphore()
        for nbr in (left, right):
            pl.semaphore_signal(
                barrier_sem, inc=1,
                device_id=(nbr,), device_id_type=pl.DeviceIdType.MESH,
            )
        pl.semaphore_wait(barrier_sem, 2)

        acc_ref[:, :] = jnp.dot(
            a_ref[:, :].astype(bf16),
            b_ref[:, :].astype(bf16),
            preferred_element_type=f32,
        ).astype(bf16)

        def rows(c):
            return pl.ds(c * cm, cm)

        for s in range(N_DEV - 1):
            cR = lax.rem(me - s + N_DEV, N_DEV)
            cL = lax.rem(me + s, N_DEV)
            slot = (s - 1) % N_DEV
            if s == 0:
                bufR[slot, :, :] = acc_ref[rows(cR), :nh]
                bufL[slot, :, :] = acc_ref[rows(cL), nh:]
            else:
                bufR[slot, :, :] = (
                    bufR[slot, :, :].astype(f32)
                    + acc_ref[rows(cR), :nh].astype(f32)
                ).astype(bf16)
                bufL[slot, :, :] = (
                    bufL[slot, :, :].astype(f32)
                    + acc_ref[rows(cL), nh:].astype(f32)
                ).astype(bf16)
            rdmaR = pltpu.make_async_remote_copy(
                src_ref=bufR.at[slot],
                dst_ref=bufR.at[s],
                send_sem=rs_sems.at[s, 0],
                recv_sem=rs_sems.at[s, 1],
                device_id=(right,),
                device_id_type=pl.DeviceIdType.MESH,
            )
            rdmaL = pltpu.make_async_remote_copy(
                src_ref=bufL.at[slot],
                dst_ref=bufL.at[s],
                send_sem=rs_sems.at[s, 2],
                recv_sem=rs_sems.at[s, 3],
                device_id=(left,),
                device_id_type=pl.DeviceIdType.MESH,
            )
            rdmaR.start()
            rdmaL.start()
            rdmaR.wait()
            rdmaL.wait()

        oR = lax.rem(me + 1, N_DEV)
        oL = lax.rem(me + (N_DEV - 1), N_DEV)

        def gelu(z):
            return 0.5 * z * (
                1.0 + jnp.tanh(0.7978845608 * (z + 0.044715 * z * z * z))
            )

        zR = (bufR[N_DEV - 2, :, :].astype(f32)
              + acc_ref[rows(oR), :nh].astype(f32))
        out_ref[rows(oR), :nh] = gelu(zR).astype(bf16)
        zL = (bufL[N_DEV - 2, :, :].astype(f32)
              + acc_ref[rows(oL), nh:].astype(f32))
        out_ref[rows(oL), nh:] = gelu(zL).astype(bf16)

        for s in range(N_DEV - 1):
            aR = lax.rem(me + 1 - s + N_DEV, N_DEV)
            aL = lax.rem(me - 1 + s + N_DEV, N_DEV)
            rdmaR = pltpu.make_async_remote_copy(
                src_ref=out_ref.at[rows(aR), :nh],
                dst_ref=out_ref.at[rows(aR), :nh],
                send_sem=ag_sems.at[s, 0],
                recv_sem=ag_sems.at[s, 1],
                device_id=(right,),
                device_id_type=pl.DeviceIdType.MESH,
            )
            rdmaL = pltpu.make_async_remote_copy(
                src_ref=out_ref.at[rows(aL), nh:],
                dst_ref=out_ref.at[rows(aL), nh:],
                send_sem=ag_sems.at[s, 2],
                recv_sem=ag_sems.at[s, 3],
                device_id=(left,),
                device_id_type=pl.DeviceIdType.MESH,
            )
            rdmaR.start()
            rdmaL.start()
            rdmaR.wait()
            rdmaL.wait()

    return pl.pallas_call(
        body,
        out_shape=jax.ShapeDtypeStruct((m, n), bf16),
        in_specs=[
            pl.BlockSpec(memory_space=pltpu.VMEM),
            pl.BlockSpec(memory_space=pltpu.VMEM),
        ],
        out_specs=pl.BlockSpec(memory_space=pltpu.VMEM),
        scratch_shapes=[
            pltpu.VMEM((m, n), bf16),
            pltpu.VMEM((N_DEV, cm, nh), bf16),
            pltpu.VMEM((N_DEV, cm, nh), bf16),
            pltpu.SemaphoreType.DMA((N_DEV - 1, 4)),
            pltpu.SemaphoreType.DMA((N_DEV - 1, 4)),
        ],
        compiler_params=pltpu.CompilerParams(
            collective_id=0,
            vmem_limit_bytes=100 * 1024 * 1024,
        ),
    )(A, B)


# device time: 156258 ns/iter; 2.1701x vs baseline; 1.3519x over previous
import jax
import jax.numpy as jnp
from jax import lax
from jax.experimental import pallas as pl
from jax.experimental.pallas import tpu as pltpu

N_DEV = 16
P_DIM = 4
Q_DIM = 4


def kernel(A, B):
    m, k = A.shape
    k2, n = B.shape
    assert k == k2
    qm = m // Q_DIM
    zm = qm // P_DIM
    nh = n // 2

    f32 = jnp.float32
    bf16 = jnp.bfloat16

    def body(a_ref, b_ref, out_ref, acc_ref, pbufR, pbufL,
             zownR, zownL, zbufR, zbufL,
             pa_sems, zr_sems, za_sems, pg_sems):
        me = lax.axis_index("i")
        p = me // Q_DIM
        q = lax.rem(me, Q_DIM)
        rightq = p * Q_DIM + lax.rem(q + 1, Q_DIM)
        leftq = p * Q_DIM + lax.rem(q + (Q_DIM - 1), Q_DIM)
        zright = lax.rem(p + 1, P_DIM) * Q_DIM + q
        zleft = lax.rem(p + (P_DIM - 1), P_DIM) * Q_DIM + q

        barrier_sem = pltpu.get_barrier_semaphore()
        for nbr in (leftq, rightq, zleft, zright):
            pl.semaphore_signal(
                barrier_sem, inc=1,
                device_id=(nbr,), device_id_type=pl.DeviceIdType.MESH,
            )
        pl.semaphore_wait(barrier_sem, 4)

        acc_ref[:, :] = jnp.dot(
            a_ref[:, :].astype(bf16),
            b_ref[:, :].astype(bf16),
            preferred_element_type=f32,
        ).astype(bf16)

        def qrows(c):
            return pl.ds(c * qm, qm)

        def zrows(c):
            return pl.ds(c * zm, zm)

        for s in range(Q_DIM - 1):
            cR = lax.rem(q - s + Q_DIM, Q_DIM)
            cL = lax.rem(q + s, Q_DIM)
            slot = (s - 1) % Q_DIM
            if s == 0:
                pbufR[slot, :, :] = acc_ref[qrows(cR), :nh]
                pbufL[slot, :, :] = acc_ref[qrows(cL), nh:]
            else:
                pbufR[slot, :, :] = (
                    pbufR[slot, :, :].astype(f32)
                    + acc_ref[qrows(cR), :nh].astype(f32)
                ).astype(bf16)
                pbufL[slot, :, :] = (
                    pbufL[slot, :, :].astype(f32)
                    + acc_ref[qrows(cL), nh:].astype(f32)
                ).astype(bf16)
            rdmaR = pltpu.make_async_remote_copy(
                src_ref=pbufR.at[slot], dst_ref=pbufR.at[s],
                send_sem=pa_sems.at[s, 0], recv_sem=pa_sems.at[s, 1],
                device_id=(rightq,), device_id_type=pl.DeviceIdType.MESH,
            )
            rdmaL = pltpu.make_async_remote_copy(
                src_ref=pbufL.at[slot], dst_ref=pbufL.at[s],
                send_sem=pa_sems.at[s, 2], recv_sem=pa_sems.at[s, 3],
                device_id=(leftq,), device_id_type=pl.DeviceIdType.MESH,
            )
            rdmaR.start()
            rdmaL.start()
            rdmaR.wait()
            rdmaL.wait()

        oRq = lax.rem(q + 1, Q_DIM)
        oLq = lax.rem(q + (Q_DIM - 1), Q_DIM)
        zownR[:, :] = (
            pbufR[Q_DIM - 2, :, :].astype(f32)
            + acc_ref[qrows(oRq), :nh].astype(f32)
        ).astype(bf16)
        zownL[:, :] = (
            pbufL[Q_DIM - 2, :, :].astype(f32)
            + acc_ref[qrows(oLq), nh:].astype(f32)
        ).astype(bf16)

        for s in range(P_DIM - 1):
            czR = lax.rem(p - s + P_DIM, P_DIM)
            czL = lax.rem(p + s, P_DIM)
            slot = (s - 1) % P_DIM
            if s == 0:
                zbufR[slot, :, :] = zownR[zrows(czR), :]
                zbufL[slot, :, :] = zownL[zrows(czL), :]
            else:
                zbufR[slot, :, :] = (
                    zbufR[slot, :, :].astype(f32)
                    + zownR[zrows(czR), :].astype(f32)
                ).astype(bf16)
                zbufL[slot, :, :] = (
                    zbufL[slot, :, :].astype(f32)
                    + zownL[zrows(czL), :].astype(f32)
                ).astype(bf16)
            rdmaR = pltpu.make_async_remote_copy(
                src_ref=zbufR.at[slot], dst_ref=zbufR.at[s],
                send_sem=zr_sems.at[s, 0], recv_sem=zr_sems.at[s, 1],
                device_id=(zright,), device_id_type=pl.DeviceIdType.MESH,
            )
            rdmaL = pltpu.make_async_remote_copy(
                src_ref=zbufL.at[slot], dst_ref=zbufL.at[s],
                send_sem=zr_sems.at[s, 2], recv_sem=zr_sems.at[s, 3],
                device_id=(zleft,), device_id_type=pl.DeviceIdType.MESH,
            )
            rdmaR.start()
            rdmaL.start()
            rdmaR.wait()
            rdmaL.wait()

        def gelu(z):
            return 0.5 * z * (
                1.0 + jnp.tanh(0.7978845608 * (z + 0.044715 * z * z * z))
            )

        ozR = lax.rem(p + 1, P_DIM)
        ozL = lax.rem(p + (P_DIM - 1), P_DIM)
        zR = (zbufR[P_DIM - 2, :, :].astype(f32)
              + zownR[zrows(ozR), :].astype(f32))
        out_ref[pl.ds(oRq * qm + ozR * zm, zm), :nh] = gelu(zR).astype(bf16)
        zL = (zbufL[P_DIM - 2, :, :].astype(f32)
              + zownL[zrows(ozL), :].astype(f32))
        out_ref[pl.ds(oLq * qm + ozL * zm, zm), nh:] = gelu(zL).astype(bf16)

        for s in range(P_DIM - 1):
            gR = lax.rem(p + 1 - s + P_DIM, P_DIM)
            gL = lax.rem(p - 1 + s + P_DIM, P_DIM)
            rdmaR = pltpu.make_async_remote_copy(
                src_ref=out_ref.at[pl.ds(oRq * qm + gR * zm, zm), :nh],
                dst_ref=out_ref.at[pl.ds(oRq * qm + gR * zm, zm), :nh],
                send_sem=za_sems.at[s, 0], recv_sem=za_sems.at[s, 1],
                device_id=(zright,), device_id_type=pl.DeviceIdType.MESH,
            )
            rdmaL = pltpu.make_async_remote_copy(
                src_ref=out_ref.at[pl.ds(oLq * qm + gL * zm, zm), nh:],
                dst_ref=out_ref.at[pl.ds(oLq * qm + gL * zm, zm), nh:],
                send_sem=za_sems.at[s, 2], recv_sem=za_sems.at[s, 3],
                device_id=(zleft,), device_id_type=pl.DeviceIdType.MESH,
            )
            rdmaR.start()
            rdmaL.start()
            rdmaR.wait()
            rdmaL.wait()

        for s in range(Q_DIM - 1):
            aR = lax.rem(q + 1 - s + Q_DIM, Q_DIM)
            aL = lax.rem(q - 1 + s + Q_DIM, Q_DIM)
            rdmaR = pltpu.make_async_remote_copy(
                src_ref=out_ref.at[qrows(aR), :nh],
                dst_ref=out_ref.at[qrows(aR), :nh],
                send_sem=pg_sems.at[s, 0], recv_sem=pg_sems.at[s, 1],
                device_id=(rightq,), device_id_type=pl.DeviceIdType.MESH,
            )
            rdmaL = pltpu.make_async_remote_copy(
                src_ref=out_ref.at[qrows(aL), nh:],
                dst_ref=out_ref.at[qrows(aL), nh:],
                send_sem=pg_sems.at[s, 2], recv_sem=pg_sems.at[s, 3],
                device_id=(leftq,), device_id_type=pl.DeviceIdType.MESH,
            )
            rdmaR.start()
            rdmaL.start()
            rdmaR.wait()
            rdmaL.wait()

    return pl.pallas_call(
        body,
        out_shape=jax.ShapeDtypeStruct((m, n), bf16),
        in_specs=[
            pl.BlockSpec(memory_space=pltpu.VMEM),
            pl.BlockSpec(memory_space=pltpu.VMEM),
        ],
        out_specs=pl.BlockSpec(memory_space=pltpu.VMEM),
        scratch_shapes=[
            pltpu.VMEM((m, n), bf16),
            pltpu.VMEM((Q_DIM, qm, nh), bf16),
            pltpu.VMEM((Q_DIM, qm, nh), bf16),
            pltpu.VMEM((qm, nh), bf16),
            pltpu.VMEM((qm, nh), bf16),
            pltpu.VMEM((P_DIM, zm, nh), bf16),
            pltpu.VMEM((P_DIM, zm, nh), bf16),
            pltpu.SemaphoreType.DMA((Q_DIM - 1, 4)),
            pltpu.SemaphoreType.DMA((P_DIM - 1, 4)),
            pltpu.SemaphoreType.DMA((P_DIM - 1, 4)),
            pltpu.SemaphoreType.DMA((Q_DIM - 1, 4)),
        ],
        compiler_params=pltpu.CompilerParams(
            collective_id=0,
            vmem_limit_bytes=100 * 1024 * 1024,
        ),
    )(A, B)


# device time: 150157 ns/iter; 2.2583x vs baseline; 1.0406x over previous
import jax
import jax.numpy as jnp
from jax import lax
from jax.experimental import pallas as pl
from jax.experimental.pallas import tpu as pltpu

N_DEV = 16
P_DIM = 4
Q_DIM = 4


def kernel(A, B):
    m, k = A.shape
    k2, n = B.shape
    assert k == k2
    qm = m // Q_DIM
    zm = qm // P_DIM
    nh = n // 2

    f32 = jnp.float32
    bf16 = jnp.bfloat16

    def body(a_ref, b_ref, out_ref, acc_ref, pbufR, pbufL,
             zownR, zownL, zbufR, zbufL,
             pa_sems, zr_sems, za_sems, pg_sems):
        me = lax.axis_index("i")
        p = me // Q_DIM
        q = lax.rem(me, Q_DIM)
        rightq = p * Q_DIM + lax.rem(q + 1, Q_DIM)
        leftq = p * Q_DIM + lax.rem(q + (Q_DIM - 1), Q_DIM)
        zright = lax.rem(p + 1, P_DIM) * Q_DIM + q
        zleft = lax.rem(p + (P_DIM - 1), P_DIM) * Q_DIM + q

        barrier_sem = pltpu.get_barrier_semaphore()
        for nbr in (leftq, rightq, zleft, zright):
            pl.semaphore_signal(
                barrier_sem, inc=1,
                device_id=(nbr,), device_id_type=pl.DeviceIdType.MESH,
            )
        pl.semaphore_wait(barrier_sem, 4)

        def qrows(c):
            return pl.ds(c * qm, qm)

        def zrows(c):
            return pl.ds(c * zm, zm)

        def mm_quarter(c):
            acc_ref[qrows(c), :] = jnp.dot(
                a_ref[qrows(c), :].astype(bf16),
                b_ref[:, :].astype(bf16),
                preferred_element_type=f32,
            ).astype(bf16)

        mm_quarter(q)

        for s in range(Q_DIM - 1):
            cR = lax.rem(q - s + Q_DIM, Q_DIM)
            cL = lax.rem(q + s, Q_DIM)
            slot = (s - 1) % Q_DIM
            if s == 0:
                pbufR[slot, :, :] = acc_ref[qrows(cR), :nh]
                pbufL[slot, :, :] = acc_ref[qrows(cL), nh:]
            else:
                pbufR[slot, :, :] = (
                    pbufR[slot, :, :].astype(f32)
                    + acc_ref[qrows(cR), :nh].astype(f32)
                ).astype(bf16)
                pbufL[slot, :, :] = (
                    pbufL[slot, :, :].astype(f32)
                    + acc_ref[qrows(cL), nh:].astype(f32)
                ).astype(bf16)
            rdmaR = pltpu.make_async_remote_copy(
                src_ref=pbufR.at[slot], dst_ref=pbufR.at[s],
                send_sem=pa_sems.at[s, 0], recv_sem=pa_sems.at[s, 1],
                device_id=(rightq,), device_id_type=pl.DeviceIdType.MESH,
            )
            rdmaL = pltpu.make_async_remote_copy(
                src_ref=pbufL.at[slot], dst_ref=pbufL.at[s],
                send_sem=pa_sems.at[s, 2], recv_sem=pa_sems.at[s, 3],
                device_id=(leftq,), device_id_type=pl.DeviceIdType.MESH,
            )
            rdmaR.start()
            rdmaL.start()
            if s == 0:
                mm_quarter(lax.rem(q + 1, Q_DIM))
                mm_quarter(lax.rem(q + (Q_DIM - 1), Q_DIM))
            elif s == 1:
                mm_quarter(lax.rem(q + 2, Q_DIM))
            rdmaR.wait()
            rdmaL.wait()

        oRq = lax.rem(q + 1, Q_DIM)
        oLq = lax.rem(q + (Q_DIM - 1), Q_DIM)
        zownR[:, :] = (
            pbufR[Q_DIM - 2, :, :].astype(f32)
            + acc_ref[qrows(oRq), :nh].astype(f32)
        ).astype(bf16)
        zownL[:, :] = (
            pbufL[Q_DIM - 2, :, :].astype(f32)
            + acc_ref[qrows(oLq), nh:].astype(f32)
        ).astype(bf16)

        for s in range(P_DIM - 1):
            czR = lax.rem(p - s + P_DIM, P_DIM)
            czL = lax.rem(p + s, P_DIM)
            slot = (s - 1) % P_DIM
            if s == 0:
                zbufR[slot, :, :] = zownR[zrows(czR), :]
                zbufL[slot, :, :] = zownL[zrows(czL), :]
            else:
                zbufR[slot, :, :] = (
                    zbufR[slot, :, :].astype(f32)
                    + zownR[zrows(czR), :].astype(f32)
                ).astype(bf16)
                zbufL[slot, :, :] = (
                    zbufL[slot, :, :].astype(f32)
                    + zownL[zrows(czL), :].astype(f32)
                ).astype(bf16)
            rdmaR = pltpu.make_async_remote_copy(
                src_ref=zbufR.at[slot], dst_ref=zbufR.at[s],
                send_sem=zr_sems.at[s, 0], recv_sem=zr_sems.at[s, 1],
                device_id=(zright,), device_id_type=pl.DeviceIdType.MESH,
            )
            rdmaL = pltpu.make_async_remote_copy(
                src_ref=zbufL.at[slot], dst_ref=zbufL.at[s],
                send_sem=zr_sems.at[s, 2], recv_sem=zr_sems.at[s, 3],
                device_id=(zleft,), device_id_type=pl.DeviceIdType.MESH,
            )
            rdmaR.start()
            rdmaL.start()
            rdmaR.wait()
            rdmaL.wait()

        def gelu(z):
            return 0.5 * z * (
                1.0 + jnp.tanh(0.7978845608 * (z + 0.044715 * z * z * z))
            )

        ozR = lax.rem(p + 1, P_DIM)
        ozL = lax.rem(p + (P_DIM - 1), P_DIM)
        zR = (zbufR[P_DIM - 2, :, :].astype(f32)
              + zownR[zrows(ozR), :].astype(f32))
        out_ref[pl.ds(oRq * qm + ozR * zm, zm), :nh] = gelu(zR).astype(bf16)
        zL = (zbufL[P_DIM - 2, :, :].astype(f32)
              + zownL[zrows(ozL), :].astype(f32))
        out_ref[pl.ds(oLq * qm + ozL * zm, zm), nh:] = gelu(zL).astype(bf16)

        for s in range(P_DIM - 1):
            gR = lax.rem(p + 1 - s + P_DIM, P_DIM)
            gL = lax.rem(p - 1 + s + P_DIM, P_DIM)
            rdmaR = pltpu.make_async_remote_copy(
                src_ref=out_ref.at[pl.ds(oRq * qm + gR * zm, zm), :nh],
                dst_ref=out_ref.at[pl.ds(oRq * qm + gR * zm, zm), :nh],
                send_sem=za_sems.at[s, 0], recv_sem=za_sems.at[s, 1],
                device_id=(zright,), device_id_type=pl.DeviceIdType.MESH,
            )
            rdmaL = pltpu.make_async_remote_copy(
                src_ref=out_ref.at[pl.ds(oLq * qm + gL * zm, zm), nh:],
                dst_ref=out_ref.at[pl.ds(oLq * qm + gL * zm, zm), nh:],
                send_sem=za_sems.at[s, 2], recv_sem=za_sems.at[s, 3],
                device_id=(zleft,), device_id_type=pl.DeviceIdType.MESH,
            )
            rdmaR.start()
            rdmaL.start()
            rdmaR.wait()
            rdmaL.wait()

        for s in range(Q_DIM - 1):
            aR = lax.rem(q + 1 - s + Q_DIM, Q_DIM)
            aL = lax.rem(q - 1 + s + Q_DIM, Q_DIM)
            rdmaR = pltpu.make_async_remote_copy(
                src_ref=out_ref.at[qrows(aR), :nh],
                dst_ref=out_ref.at[qrows(aR), :nh],
                send_sem=pg_sems.at[s, 0], recv_sem=pg_sems.at[s, 1],
                device_id=(rightq,), device_id_type=pl.DeviceIdType.MESH,
            )
            rdmaL = pltpu.make_async_remote_copy(
                src_ref=out_ref.at[qrows(aL), nh:],
                dst_ref=out_ref.at[qrows(aL), nh:],
                send_sem=pg_sems.at[s, 2], recv_sem=pg_sems.at[s, 3],
                device_id=(leftq,), device_id_type=pl.DeviceIdType.MESH,
            )
            rdmaR.start()
            rdmaL.start()
            rdmaR.wait()
            rdmaL.wait()

    return pl.pallas_call(
        body,
        out_shape=jax.ShapeDtypeStruct((m, n), bf16),
        in_specs=[
            pl.BlockSpec(memory_space=pltpu.VMEM),
            pl.BlockSpec(memory_space=pltpu.VMEM),
        ],
        out_specs=pl.BlockSpec(memory_space=pltpu.VMEM),
        scratch_shapes=[
            pltpu.VMEM((m, n), bf16),
            pltpu.VMEM((Q_DIM, qm, nh), bf16),
            pltpu.VMEM((Q_DIM, qm, nh), bf16),
            pltpu.VMEM((qm, nh), bf16),
            pltpu.VMEM((qm, nh), bf16),
            pltpu.VMEM((P_DIM, zm, nh), bf16),
            pltpu.VMEM((P_DIM, zm, nh), bf16),
            pltpu.SemaphoreType.DMA((Q_DIM - 1, 4)),
            pltpu.SemaphoreType.DMA((P_DIM - 1, 4)),
            pltpu.SemaphoreType.DMA((P_DIM - 1, 4)),
            pltpu.SemaphoreType.DMA((Q_DIM - 1, 4)),
        ],
        compiler_params=pltpu.CompilerParams(
            collective_id=0,
            vmem_limit_bytes=100 * 1024 * 1024,
        ),
    )(A, B)


# device time: 138052 ns/iter; 2.4563x vs baseline; 1.0877x over previous
import jax
import jax.numpy as jnp
from jax import lax
from jax.experimental import pallas as pl
from jax.experimental.pallas import tpu as pltpu

N_DEV = 16
P_DIM = 4
Q_DIM = 4


def kernel(A, B):
    m, k = A.shape
    k2, n = B.shape
    assert k == k2
    qm = m // Q_DIM
    zm = qm // P_DIM
    nh = n // 2

    f32 = jnp.float32
    bf16 = jnp.bfloat16

    def body(a_ref, b_ref, out_ref, acc_ref, pbufR, pbufL,
             zbufR, zbufL,
             pa_sems, zr_sems, za_sems, pg_sems):
        me = lax.axis_index("i")
        p = me // Q_DIM
        q = lax.rem(me, Q_DIM)
        rightq = p * Q_DIM + lax.rem(q + 1, Q_DIM)
        leftq = p * Q_DIM + lax.rem(q + (Q_DIM - 1), Q_DIM)
        zright = lax.rem(p + 1, P_DIM) * Q_DIM + q
        zleft = lax.rem(p + (P_DIM - 1), P_DIM) * Q_DIM + q

        barrier_sem = pltpu.get_barrier_semaphore()
        for nbr in (leftq, rightq, zleft, zright):
            pl.semaphore_signal(
                barrier_sem, inc=1,
                device_id=(nbr,), device_id_type=pl.DeviceIdType.MESH,
            )
        pl.semaphore_wait(barrier_sem, 4)

        def qrows(c):
            return pl.ds(c * qm, qm)

        def zrows(c):
            return pl.ds(c * zm, zm)

        def mm_quarter(c):
            acc_ref[qrows(c), :] = jnp.dot(
                a_ref[qrows(c), :].astype(bf16),
                b_ref[:, :].astype(bf16),
                preferred_element_type=f32,
            ).astype(bf16)

        mm_quarter(q)

        for s in range(Q_DIM - 1):
            cR = lax.rem(q - s + Q_DIM, Q_DIM)
            cL = lax.rem(q + s, Q_DIM)
            slot = (s - 1) % Q_DIM
            if s == 0:
                pbufR[slot, :, :] = acc_ref[qrows(cR), :nh]
            else:
                pbufR[slot, :, :] = (
                    pbufR[slot, :, :].astype(f32)
                    + acc_ref[qrows(cR), :nh].astype(f32)
                ).astype(bf16)
            rdmaR = pltpu.make_async_remote_copy(
                src_ref=pbufR.at[slot], dst_ref=pbufR.at[s],
                send_sem=pa_sems.at[s, 0], recv_sem=pa_sems.at[s, 1],
                device_id=(rightq,), device_id_type=pl.DeviceIdType.MESH,
            )
            rdmaR.start()
            if s == 0:
                pbufL[slot, :, :] = acc_ref[qrows(cL), nh:]
            else:
                pbufL[slot, :, :] = (
                    pbufL[slot, :, :].astype(f32)
                    + acc_ref[qrows(cL), nh:].astype(f32)
                ).astype(bf16)
            rdmaL = pltpu.make_async_remote_copy(
                src_ref=pbufL.at[slot], dst_ref=pbufL.at[s],
                send_sem=pa_sems.at[s, 2], recv_sem=pa_sems.at[s, 3],
                device_id=(leftq,), device_id_type=pl.DeviceIdType.MESH,
            )
            rdmaL.start()
            if s == 0:
                mm_quarter(lax.rem(q + 1, Q_DIM))
                mm_quarter(lax.rem(q + (Q_DIM - 1), Q_DIM))
            elif s == 1:
                mm_quarter(lax.rem(q + 2, Q_DIM))
            rdmaR.wait()
            rdmaL.wait()

        oRq = lax.rem(q + 1, Q_DIM)
        oLq = lax.rem(q + (Q_DIM - 1), Q_DIM)

        def zown_R(c):
            return (pbufR[Q_DIM - 2, zrows(c), :].astype(f32)
                    + acc_ref[pl.ds(oRq * qm + c * zm, zm), :nh].astype(f32))

        def zown_L(c):
            return (pbufL[Q_DIM - 2, zrows(c), :].astype(f32)
                    + acc_ref[pl.ds(oLq * qm + c * zm, zm), nh:].astype(f32))

        for s in range(P_DIM - 1):
            czR = lax.rem(p - s + P_DIM, P_DIM)
            czL = lax.rem(p + s, P_DIM)
            slot = (s - 1) % P_DIM
            if s == 0:
                zbufR[slot, :, :] = zown_R(czR).astype(bf16)
            else:
                zbufR[slot, :, :] = (
                    zbufR[slot, :, :].astype(f32) + zown_R(czR)
                ).astype(bf16)
            rdmaR = pltpu.make_async_remote_copy(
                src_ref=zbufR.at[slot], dst_ref=zbufR.at[s],
                send_sem=zr_sems.at[s, 0], recv_sem=zr_sems.at[s, 1],
                device_id=(zright,), device_id_type=pl.DeviceIdType.MESH,
            )
            rdmaR.start()
            if s == 0:
                zbufL[slot, :, :] = zown_L(czL).astype(bf16)
            else:
                zbufL[slot, :, :] = (
                    zbufL[slot, :, :].astype(f32) + zown_L(czL)
                ).astype(bf16)
            rdmaL = pltpu.make_async_remote_copy(
                src_ref=zbufL.at[slot], dst_ref=zbufL.at[s],
                send_sem=zr_sems.at[s, 2], recv_sem=zr_sems.at[s, 3],
                device_id=(zleft,), device_id_type=pl.DeviceIdType.MESH,
            )
            rdmaL.start()
            rdmaR.wait()
            rdmaL.wait()

        def gelu(z):
            return 0.5 * z * (
                1.0 + jnp.tanh(0.7978845608 * (z + 0.044715 * z * z * z))
            )

        ozR = lax.rem(p + 1, P_DIM)
        ozL = lax.rem(p + (P_DIM - 1), P_DIM)
        zR = zbufR[P_DIM - 2, :, :].astype(f32) + zown_R(ozR)
        out_ref[pl.ds(oRq * qm + ozR * zm, zm), :nh] = gelu(zR).astype(bf16)
        zL = zbufL[P_DIM - 2, :, :].astype(f32) + zown_L(ozL)
        out_ref[pl.ds(oLq * qm + ozL * zm, zm), nh:] = gelu(zL).astype(bf16)

        def z_hop(s):
            gR = lax.rem(p + 1 - s + P_DIM, P_DIM)
            gL = lax.rem(p - 1 + s + P_DIM, P_DIM)
            dR = pltpu.make_async_remote_copy(
                src_ref=out_ref.at[pl.ds(oRq * qm + gR * zm, zm), :nh],
                dst_ref=out_ref.at[pl.ds(oRq * qm + gR * zm, zm), :nh],
                send_sem=za_sems.at[s, 0], recv_sem=za_sems.at[s, 1],
                device_id=(zright,), device_id_type=pl.DeviceIdType.MESH,
            )
            dL = pltpu.make_async_remote_copy(
                src_ref=out_ref.at[pl.ds(oLq * qm + gL * zm, zm), nh:],
                dst_ref=out_ref.at[pl.ds(oLq * qm + gL * zm, zm), nh:],
                send_sem=za_sems.at[s, 2], recv_sem=za_sems.at[s, 3],
                device_id=(zleft,), device_id_type=pl.DeviceIdType.MESH,
            )
            dR.start()
            dL.start()
            return (dR, dL)

        def plane_hop(r, h):
            czR_ = lax.rem(p + 1 - r + P_DIM, P_DIM)
            qR_ = lax.rem(q + 1 - h + Q_DIM, Q_DIM)
            rowsR = pl.ds(qR_ * qm + czR_ * zm, zm)
            dR = pltpu.make_async_remote_copy(
                src_ref=out_ref.at[rowsR, :nh],
                dst_ref=out_ref.at[rowsR, :nh],
                send_sem=pg_sems.at[r, h, 0], recv_sem=pg_sems.at[r, h, 1],
                device_id=(rightq,), device_id_type=pl.DeviceIdType.MESH,
            )
            czL_ = lax.rem(p - 1 + r + P_DIM, P_DIM)
            qL_ = lax.rem(q - 1 + h + Q_DIM, Q_DIM)
            rowsL = pl.ds(qL_ * qm + czL_ * zm, zm)
            dL = pltpu.make_async_remote_copy(
                src_ref=out_ref.at[rowsL, nh:],
                dst_ref=out_ref.at[rowsL, nh:],
                send_sem=pg_sems.at[r, h, 2], recv_sem=pg_sems.at[r, h, 3],
                device_id=(leftq,), device_id_type=pl.DeviceIdType.MESH,
            )
            dR.start()
            dL.start()
            return (dR, dL)

        zd = {}
        pr = {}
        zd[0] = z_hop(0)
        pr[(0, 0)] = plane_hop(0, 0)
        for s in (1, 2):
            zd[s - 1][0].wait_recv()
            zd[s - 1][1].wait_recv()
            zd[s] = z_hop(s)
            pr[(s, 0)] = plane_hop(s, 0)
        zd[2][0].wait_recv()
        zd[2][1].wait_recv()
        pr[(3, 0)] = plane_hop(3, 0)
        for h in (1, 2):
            for r in range(P_DIM):
                pr[(r, h - 1)][0].wait_recv()
                pr[(r, h - 1)][1].wait_recv()
                pr[(r, h)] = plane_hop(r, h)
        for r in range(P_DIM):
            pr[(r, 2)][0].wait_recv()
            pr[(r, 2)][1].wait_recv()
        for pair in list(zd.values()) + list(pr.values()):
            pair[0].wait_send()
            pair[1].wait_send()

    return pl.pallas_call(
        body,
        out_shape=jax.ShapeDtypeStruct((m, n), bf16),
        in_specs=[
            pl.BlockSpec(memory_space=pltpu.VMEM),
            pl.BlockSpec(memory_space=pltpu.VMEM),
        ],
        out_specs=pl.BlockSpec(memory_space=pltpu.VMEM),
        scratch_shapes=[
            pltpu.VMEM((m, n), bf16),
            pltpu.VMEM((Q_DIM, qm, nh), bf16),
            pltpu.VMEM((Q_DIM, qm, nh), bf16),
            pltpu.VMEM((P_DIM, zm, nh), bf16),
            pltpu.VMEM((P_DIM, zm, nh), bf16),
            pltpu.SemaphoreType.DMA((Q_DIM - 1, 4)),
            pltpu.SemaphoreType.DMA((P_DIM - 1, 4)),
            pltpu.SemaphoreType.DMA((P_DIM - 1, 4)),
            pltpu.SemaphoreType.DMA((P_DIM, Q_DIM - 1, 4)),
        ],
        compiler_params=pltpu.CompilerParams(
            collective_id=0,
            vmem_limit_bytes=100 * 1024 * 1024,
        ),
    )(A, B)


# device time: 121259 ns/iter; 2.7965x vs baseline; 1.1385x over previous
import jax
import jax.numpy as jnp
from jax import lax
from jax.experimental import pallas as pl
from jax.experimental.pallas import tpu as pltpu

N_DEV = 16
P_DIM = 4
Q_DIM = 4


def kernel(A, B):
    m, k = A.shape
    k2, n = B.shape
    assert k == k2
    qm = m // Q_DIM
    zm = qm // P_DIM
    nh = n // 2

    f32 = jnp.float32
    bf16 = jnp.bfloat16

    def body(a_ref, b_ref, out_ref, acc_ref, pbufR, pbufL,
             zbufR, zbufL,
             pa_sems, zr_sems, za_sems, pg_sems):
        me = lax.axis_index("i")
        p = me // Q_DIM
        q = lax.rem(me, Q_DIM)
        rightq = p * Q_DIM + lax.rem(q + 1, Q_DIM)
        leftq = p * Q_DIM + lax.rem(q + (Q_DIM - 1), Q_DIM)
        zright = lax.rem(p + 1, P_DIM) * Q_DIM + q
        zleft = lax.rem(p + (P_DIM - 1), P_DIM) * Q_DIM + q

        barrier_sem = pltpu.get_barrier_semaphore()
        for nbr in (leftq, rightq, zleft, zright):
            pl.semaphore_signal(
                barrier_sem, inc=1,
                device_id=(nbr,), device_id_type=pl.DeviceIdType.MESH,
            )
        pl.semaphore_wait(barrier_sem, 4)

        def qrows(c):
            return pl.ds(c * qm, qm)

        def zrows(c):
            return pl.ds(c * zm, zm)

        def mm_quarter(c):
            acc_ref[qrows(c), :] = jnp.dot(
                a_ref[qrows(c), :].astype(bf16),
                b_ref[:, :].astype(bf16),
                preferred_element_type=f32,
            ).astype(bf16)

        oRq = lax.rem(q + 1, Q_DIM)
        oLq = lax.rem(q + (Q_DIM - 1), Q_DIM)
        ozR = lax.rem(p + 1, P_DIM)
        ozL = lax.rem(p + (P_DIM - 1), P_DIM)

        def tR(j):
            if j < P_DIM - 1:
                return lax.rem(p - j + P_DIM, P_DIM)
            return lax.rem(p + 1, P_DIM)

        def tL(j):
            if j < P_DIM - 1:
                return lax.rem(p + j, P_DIM)
            return lax.rem(p + (P_DIM - 1), P_DIM)

        mm_quarter(q)
        prs = {}

        def plane_rs_hop(d, j, h, src_slot, t):
            if d == "R":
                desc = pltpu.make_async_remote_copy(
                    src_ref=pbufR.at[src_slot, zrows(t), :],
                    dst_ref=pbufR.at[h, zrows(t), :],
                    send_sem=pa_sems.at[j, h, 0], recv_sem=pa_sems.at[j, h, 1],
                    device_id=(rightq,), device_id_type=pl.DeviceIdType.MESH,
                )
            else:
                desc = pltpu.make_async_remote_copy(
                    src_ref=pbufL.at[src_slot, zrows(t), :],
                    dst_ref=pbufL.at[h, zrows(t), :],
                    send_sem=pa_sems.at[j, h, 2], recv_sem=pa_sems.at[j, h, 3],
                    device_id=(leftq,), device_id_type=pl.DeviceIdType.MESH,
                )
            desc.start()
            prs[(d, j, h)] = desc

        pbufR[Q_DIM - 1, :, :] = acc_ref[qrows(q), :nh]
        pbufL[Q_DIM - 1, :, :] = acc_ref[qrows(q), nh:]
        for j in range(P_DIM):
            plane_rs_hop("R", j, 0, Q_DIM - 1, tR(j))
            plane_rs_hop("L", j, 0, Q_DIM - 1, tL(j))
        mm_quarter(lax.rem(q + 1, Q_DIM))
        mm_quarter(lax.rem(q + (Q_DIM - 1), Q_DIM))

        for h in (1, 2):
            qR_h = lax.rem(q - h + Q_DIM, Q_DIM)
            qL_h = lax.rem(q + h, Q_DIM)
            for j in range(P_DIM):
                t = tR(j)
                prs[("R", j, h - 1)].wait_recv()
                pbufR[h - 1, zrows(t), :] = (
                    pbufR[h - 1, zrows(t), :].astype(f32)
                    + acc_ref[pl.ds(qR_h * qm + t * zm, zm), :nh].astype(f32)
                ).astype(bf16)
                plane_rs_hop("R", j, h, h - 1, t)
                t = tL(j)
                prs[("L", j, h - 1)].wait_recv()
                pbufL[h - 1, zrows(t), :] = (
                    pbufL[h - 1, zrows(t), :].astype(f32)
                    + acc_ref[pl.ds(qL_h * qm + t * zm, zm), nh:].astype(f32)
                ).astype(bf16)
                plane_rs_hop("L", j, h, h - 1, t)
            if h == 1:
                mm_quarter(lax.rem(q + 2, Q_DIM))

        def plane_final_R(c):
            return (pbufR[Q_DIM - 2, zrows(c), :].astype(f32)
                    + acc_ref[pl.ds(oRq * qm + c * zm, zm), :nh].astype(f32))

        def plane_final_L(c):
            return (pbufL[Q_DIM - 2, zrows(c), :].astype(f32)
                    + acc_ref[pl.ds(oLq * qm + c * zm, zm), nh:].astype(f32))

        zrs = {}

        def z_rs_hop(d, s, src_slot):
            if d == "R":
                desc = pltpu.make_async_remote_copy(
                    src_ref=zbufR.at[src_slot], dst_ref=zbufR.at[s],
                    send_sem=zr_sems.at[s, 0], recv_sem=zr_sems.at[s, 1],
                    device_id=(zright,), device_id_type=pl.DeviceIdType.MESH,
                )
            else:
                desc = pltpu.make_async_remote_copy(
                    src_ref=zbufL.at[src_slot], dst_ref=zbufL.at[s],
                    send_sem=zr_sems.at[s, 2], recv_sem=zr_sems.at[s, 3],
                    device_id=(zleft,), device_id_type=pl.DeviceIdType.MESH,
                )
            desc.start()
            zrs[(d, s)] = desc

        prs[("R", 0, 2)].wait_recv()
        zbufR[P_DIM - 1, :, :] = plane_final_R(tR(0)).astype(bf16)
        z_rs_hop("R", 0, P_DIM - 1)
        prs[("L", 0, 2)].wait_recv()
        zbufL[P_DIM - 1, :, :] = plane_final_L(tL(0)).astype(bf16)
        z_rs_hop("L", 0, P_DIM - 1)
        for s in (1, 2):
            zrs[("R", s - 1)].wait_recv()
            prs[("R", s, 2)].wait_recv()
            zbufR[s - 1, :, :] = (
                zbufR[s - 1, :, :].astype(f32) + plane_final_R(tR(s))
            ).astype(bf16)
            z_rs_hop("R", s, s - 1)
            zrs[("L", s - 1)].wait_recv()
            prs[("L", s, 2)].wait_recv()
            zbufL[s - 1, :, :] = (
                zbufL[s - 1, :, :].astype(f32) + plane_final_L(tL(s))
            ).astype(bf16)
            z_rs_hop("L", s, s - 1)

        def gelu(z):
            return 0.5 * z * (
                1.0 + jnp.tanh(0.7978845608 * (z + 0.044715 * z * z * z))
            )

        zrs[("R", 2)].wait_recv()
        prs[("R", 3, 2)].wait_recv()
        zR = zbufR[P_DIM - 2, :, :].astype(f32) + plane_final_R(ozR)
        out_ref[pl.ds(oRq * qm + ozR * zm, zm), :nh] = gelu(zR).astype(bf16)
        zrs[("L", 2)].wait_recv()
        prs[("L", 3, 2)].wait_recv()
        zL = zbufL[P_DIM - 2, :, :].astype(f32) + plane_final_L(ozL)
        out_ref[pl.ds(oLq * qm + ozL * zm, zm), nh:] = gelu(zL).astype(bf16)

        def z_hop(s):
            gR = lax.rem(p + 1 - s + P_DIM, P_DIM)
            gL = lax.rem(p - 1 + s + P_DIM, P_DIM)
            dR = pltpu.make_async_remote_copy(
                src_ref=out_ref.at[pl.ds(oRq * qm + gR * zm, zm), :nh],
                dst_ref=out_ref.at[pl.ds(oRq * qm + gR * zm, zm), :nh],
                send_sem=za_sems.at[s, 0], recv_sem=za_sems.at[s, 1],
                device_id=(zright,), device_id_type=pl.DeviceIdType.MESH,
            )
            dL = pltpu.make_async_remote_copy(
                src_ref=out_ref.at[pl.ds(oLq * qm + gL * zm, zm), nh:],
                dst_ref=out_ref.at[pl.ds(oLq * qm + gL * zm, zm), nh:],
                send_sem=za_sems.at[s, 2], recv_sem=za_sems.at[s, 3],
                device_id=(zleft,), device_id_type=pl.DeviceIdType.MESH,
            )
            dR.start()
            dL.start()
            return (dR, dL)

        def plane_hop(r, h):
            czR_ = lax.rem(p + 1 - r + P_DIM, P_DIM)
            qR_ = lax.rem(q + 1 - h + Q_DIM, Q_DIM)
            rowsR = pl.ds(qR_ * qm + czR_ * zm, zm)
            dR = pltpu.make_async_remote_copy(
                src_ref=out_ref.at[rowsR, :nh],
                dst_ref=out_ref.at[rowsR, :nh],
                send_sem=pg_sems.at[r, h, 0], recv_sem=pg_sems.at[r, h, 1],
                device_id=(rightq,), device_id_type=pl.DeviceIdType.MESH,
            )
            czL_ = lax.rem(p - 1 + r + P_DIM, P_DIM)
            qL_ = lax.rem(q - 1 + h + Q_DIM, Q_DIM)
            rowsL = pl.ds(qL_ * qm + czL_ * zm, zm)
            dL = pltpu.make_async_remote_copy(
                src_ref=out_ref.at[rowsL, nh:],
                dst_ref=out_ref.at[rowsL, nh:],
                send_sem=pg_sems.at[r, h, 2], recv_sem=pg_sems.at[r, h, 3],
                device_id=(leftq,), device_id_type=pl.DeviceIdType.MESH,
            )
            dR.start()
            dL.start()
            return (dR, dL)

        zd = {}
        pr = {}
        zd[0] = z_hop(0)
        pr[(0, 0)] = plane_hop(0, 0)
        for s in (1, 2):
            zd[s - 1][0].wait_recv()
            zd[s - 1][1].wait_recv()
            zd[s] = z_hop(s)
            pr[(s, 0)] = plane_hop(s, 0)
        zd[2][0].wait_recv()
        zd[2][1].wait_recv()
        pr[(3, 0)] = plane_hop(3, 0)
        for h in (1, 2):
            for r in range(P_DIM):
                pr[(r, h - 1)][0].wait_recv()
                pr[(r, h - 1)][1].wait_recv()
                pr[(r, h)] = plane_hop(r, h)
        for r in range(P_DIM):
            pr[(r, 2)][0].wait_recv()
            pr[(r, 2)][1].wait_recv()
        for pair in list(zd.values()) + list(pr.values()):
            pair[0].wait_send()
            pair[1].wait_send()
        for desc in list(prs.values()) + list(zrs.values()):
            desc.wait_send()

    return pl.pallas_call(
        body,
        out_shape=jax.ShapeDtypeStruct((m, n), bf16),
        in_specs=[
            pl.BlockSpec(memory_space=pltpu.VMEM),
            pl.BlockSpec(memory_space=pltpu.VMEM),
        ],
        out_specs=pl.BlockSpec(memory_space=pltpu.VMEM),
        scratch_shapes=[
            pltpu.VMEM((m, n), bf16),
            pltpu.VMEM((Q_DIM, qm, nh), bf16),
            pltpu.VMEM((Q_DIM, qm, nh), bf16),
            pltpu.VMEM((P_DIM, zm, nh), bf16),
            pltpu.VMEM((P_DIM, zm, nh), bf16),
            pltpu.SemaphoreType.DMA((P_DIM, Q_DIM - 1, 4)),
            pltpu.SemaphoreType.DMA((P_DIM - 1, 4)),
            pltpu.SemaphoreType.DMA((P_DIM - 1, 4)),
            pltpu.SemaphoreType.DMA((P_DIM, Q_DIM - 1, 4)),
        ],
        compiler_params=pltpu.CompilerParams(
            collective_id=0,
            vmem_limit_bytes=100 * 1024 * 1024,
        ),
    )(A, B)


# device time: 113541 ns/iter; 2.9866x vs baseline; 1.0680x over previous
import jax
import jax.numpy as jnp
from jax import lax
from jax.experimental import pallas as pl
from jax.experimental.pallas import tpu as pltpu

N_DEV = 16
P_DIM = 4
Q_DIM = 4


def kernel(A, B):
    m, k = A.shape
    k2, n = B.shape
    assert k == k2
    qm = m // Q_DIM
    zm = qm // P_DIM
    nh = n // 2

    f32 = jnp.float32
    bf16 = jnp.bfloat16

    def body(a_ref, b_ref, out_ref, acc_ref, pbufR, pbufL,
             zbufR, zbufL,
             pa_sems, zr_sems, za_sems, pg_sems):
        me = lax.axis_index("i")
        p = me // Q_DIM
        q = lax.rem(me, Q_DIM)
        rightq = p * Q_DIM + lax.rem(q + 1, Q_DIM)
        leftq = p * Q_DIM + lax.rem(q + (Q_DIM - 1), Q_DIM)
        zright = lax.rem(p + 1, P_DIM) * Q_DIM + q
        zleft = lax.rem(p + (P_DIM - 1), P_DIM) * Q_DIM + q

        barrier_sem = pltpu.get_barrier_semaphore()
        for nbr in (leftq, rightq, zleft, zright):
            pl.semaphore_signal(
                barrier_sem, inc=1,
                device_id=(nbr,), device_id_type=pl.DeviceIdType.MESH,
            )
        pl.semaphore_wait(barrier_sem, 4)

        def qrows(c):
            return pl.ds(c * qm, qm)

        def zrows(c):
            return pl.ds(c * zm, zm)

        def mm_quarter(c):
            acc_ref[qrows(c), :] = jnp.dot(
                a_ref[qrows(c), :].astype(bf16),
                b_ref[:, :].astype(bf16),
                preferred_element_type=f32,
            ).astype(bf16)

        oRq = lax.rem(q + 1, Q_DIM)
        oLq = lax.rem(q + (Q_DIM - 1), Q_DIM)
        ozR = lax.rem(p + 1, P_DIM)
        ozL = lax.rem(p + (P_DIM - 1), P_DIM)

        def tR(j):
            if j < P_DIM - 1:
                return lax.rem(p - j + P_DIM, P_DIM)
            return lax.rem(p + 1, P_DIM)

        def tL(j):
            if j < P_DIM - 1:
                return lax.rem(p + j, P_DIM)
            return lax.rem(p + (P_DIM - 1), P_DIM)

        def mm_slab(cq, cz):
            rows = pl.ds(cq * qm + cz * zm, zm)
            acc_ref[rows, :] = jnp.dot(
                a_ref[rows, :].astype(bf16),
                b_ref[:, :].astype(bf16),
                preferred_element_type=f32,
            ).astype(bf16)

        prs = {}

        def plane_rs_hop(d, j, h, src, t):
            if d == "R":
                src_ref = (acc_ref.at[pl.ds(q * qm + t * zm, zm), :nh]
                           if src is None else pbufR.at[src, zrows(t), :])
                desc = pltpu.make_async_remote_copy(
                    src_ref=src_ref,
                    dst_ref=pbufR.at[h, zrows(t), :],
                    send_sem=pa_sems.at[j, h, 0], recv_sem=pa_sems.at[j, h, 1],
                    device_id=(rightq,), device_id_type=pl.DeviceIdType.MESH,
                )
            else:
                src_ref = (acc_ref.at[pl.ds(q * qm + t * zm, zm), nh:]
                           if src is None else pbufL.at[src, zrows(t), :])
                desc = pltpu.make_async_remote_copy(
                    src_ref=src_ref,
                    dst_ref=pbufL.at[h, zrows(t), :],
                    send_sem=pa_sems.at[j, h, 2], recv_sem=pa_sems.at[j, h, 3],
                    device_id=(leftq,), device_id_type=pl.DeviceIdType.MESH,
                )
            desc.start()
            prs[(d, j, h)] = desc

        mm_slab(q, lax.rem(p, P_DIM))
        plane_rs_hop("R", 0, 0, None, tR(0))
        plane_rs_hop("L", 0, 0, None, tL(0))
        mm_slab(q, lax.rem(p + (P_DIM - 1), P_DIM))
        plane_rs_hop("R", 1, 0, None, tR(1))
        mm_slab(q, lax.rem(p + 1, P_DIM))
        plane_rs_hop("L", 1, 0, None, tL(1))
        mm_slab(q, lax.rem(p + 2, P_DIM))
        plane_rs_hop("R", 2, 0, None, tR(2))
        plane_rs_hop("L", 2, 0, None, tL(2))
        plane_rs_hop("R", 3, 0, None, tR(3))
        plane_rs_hop("L", 3, 0, None, tL(3))
        mm_quarter(lax.rem(q + 1, Q_DIM))
        mm_quarter(lax.rem(q + (Q_DIM - 1), Q_DIM))

        for h in (1, 2):
            qR_h = lax.rem(q - h + Q_DIM, Q_DIM)
            qL_h = lax.rem(q + h, Q_DIM)
            for j in range(P_DIM):
                t = tR(j)
                prs[("R", j, h - 1)].wait_recv()
                pbufR[h - 1, zrows(t), :] = (
                    pbufR[h - 1, zrows(t), :].astype(f32)
                    + acc_ref[pl.ds(qR_h * qm + t * zm, zm), :nh].astype(f32)
                ).astype(bf16)
                plane_rs_hop("R", j, h, h - 1, t)
                t = tL(j)
                prs[("L", j, h - 1)].wait_recv()
                pbufL[h - 1, zrows(t), :] = (
                    pbufL[h - 1, zrows(t), :].astype(f32)
                    + acc_ref[pl.ds(qL_h * qm + t * zm, zm), nh:].astype(f32)
                ).astype(bf16)
                plane_rs_hop("L", j, h, h - 1, t)
            if h == 1:
                mm_quarter(lax.rem(q + 2, Q_DIM))

        def plane_final_R(c):
            return (pbufR[Q_DIM - 2, zrows(c), :].astype(f32)
                    + acc_ref[pl.ds(oRq * qm + c * zm, zm), :nh].astype(f32))

        def plane_final_L(c):
            return (pbufL[Q_DIM - 2, zrows(c), :].astype(f32)
                    + acc_ref[pl.ds(oLq * qm + c * zm, zm), nh:].astype(f32))

        zrs = {}

        def z_rs_hop(d, s, src_slot):
            if d == "R":
                desc = pltpu.make_async_remote_copy(
                    src_ref=zbufR.at[src_slot], dst_ref=zbufR.at[s],
                    send_sem=zr_sems.at[s, 0], recv_sem=zr_sems.at[s, 1],
                    device_id=(zright,), device_id_type=pl.DeviceIdType.MESH,
                )
            else:
                desc = pltpu.make_async_remote_copy(
                    src_ref=zbufL.at[src_slot], dst_ref=zbufL.at[s],
                    send_sem=zr_sems.at[s, 2], recv_sem=zr_sems.at[s, 3],
                    device_id=(zleft,), device_id_type=pl.DeviceIdType.MESH,
                )
            desc.start()
            zrs[(d, s)] = desc

        prs[("R", 0, 2)].wait_recv()
        zbufR[P_DIM - 1, :, :] = plane_final_R(tR(0)).astype(bf16)
        z_rs_hop("R", 0, P_DIM - 1)
        prs[("L", 0, 2)].wait_recv()
        zbufL[P_DIM - 1, :, :] = plane_final_L(tL(0)).astype(bf16)
        z_rs_hop("L", 0, P_DIM - 1)
        for s in (1, 2):
            zrs[("R", s - 1)].wait_recv()
            prs[("R", s, 2)].wait_recv()
            zbufR[s - 1, :, :] = (
                zbufR[s - 1, :, :].astype(f32) + plane_final_R(tR(s))
            ).astype(bf16)
            z_rs_hop("R", s, s - 1)
            zrs[("L", s - 1)].wait_recv()
            prs[("L", s, 2)].wait_recv()
            zbufL[s - 1, :, :] = (
                zbufL[s - 1, :, :].astype(f32) + plane_final_L(tL(s))
            ).astype(bf16)
            z_rs_hop("L", s, s - 1)

        def gelu(z):
            return 0.5 * z * (
                1.0 + jnp.tanh(0.7978845608 * (z + 0.044715 * z * z * z))
            )

        def z_hop(d, s):
            if d == "R":
                g = lax.rem(p + 1 - s + P_DIM, P_DIM)
                rows = pl.ds(oRq * qm + g * zm, zm)
                desc = pltpu.make_async_remote_copy(
                    src_ref=out_ref.at[rows, :nh],
                    dst_ref=out_ref.at[rows, :nh],
                    send_sem=za_sems.at[s, 0], recv_sem=za_sems.at[s, 1],
                    device_id=(zright,), device_id_type=pl.DeviceIdType.MESH,
                )
            else:
                g = lax.rem(p - 1 + s + P_DIM, P_DIM)
                rows = pl.ds(oLq * qm + g * zm, zm)
                desc = pltpu.make_async_remote_copy(
                    src_ref=out_ref.at[rows, nh:],
                    dst_ref=out_ref.at[rows, nh:],
                    send_sem=za_sems.at[s, 2], recv_sem=za_sems.at[s, 3],
                    device_id=(zleft,), device_id_type=pl.DeviceIdType.MESH,
                )
            desc.start()
            return desc

        def plane_hop(d, r, h):
            if d == "R":
                cz = lax.rem(p + 1 - r + P_DIM, P_DIM)
                qh = lax.rem(q + 1 - h + Q_DIM, Q_DIM)
                rows = pl.ds(qh * qm + cz * zm, zm)
                desc = pltpu.make_async_remote_copy(
                    src_ref=out_ref.at[rows, :nh],
                    dst_ref=out_ref.at[rows, :nh],
                    send_sem=pg_sems.at[r, h, 0], recv_sem=pg_sems.at[r, h, 1],
                    device_id=(rightq,), device_id_type=pl.DeviceIdType.MESH,
                )
            else:
                cz = lax.rem(p - 1 + r + P_DIM, P_DIM)
                qh = lax.rem(q - 1 + h + Q_DIM, Q_DIM)
                rows = pl.ds(qh * qm + cz * zm, zm)
                desc = pltpu.make_async_remote_copy(
                    src_ref=out_ref.at[rows, nh:],
                    dst_ref=out_ref.at[rows, nh:],
                    send_sem=pg_sems.at[r, h, 2], recv_sem=pg_sems.at[r, h, 3],
                    device_id=(leftq,), device_id_type=pl.DeviceIdType.MESH,
                )
            desc.start()
            return desc

        zd = {}
        pr = {}
        zrs[("R", 2)].wait_recv()
        prs[("R", 3, 2)].wait_recv()
        zR = zbufR[P_DIM - 2, :, :].astype(f32) + plane_final_R(ozR)
        out_ref[pl.ds(oRq * qm + ozR * zm, zm), :nh] = gelu(zR).astype(bf16)
        zd[(0, "R")] = z_hop("R", 0)
        pr[(0, 0, "R")] = plane_hop("R", 0, 0)
        zrs[("L", 2)].wait_recv()
        prs[("L", 3, 2)].wait_recv()
        zL = zbufL[P_DIM - 2, :, :].astype(f32) + plane_final_L(ozL)
        out_ref[pl.ds(oLq * qm + ozL * zm, zm), nh:] = gelu(zL).astype(bf16)
        zd[(0, "L")] = z_hop("L", 0)
        pr[(0, 0, "L")] = plane_hop("L", 0, 0)
        for s in (1, 2):
            for d in ("R", "L"):
                zd[(s - 1, d)].wait_recv()
                zd[(s, d)] = z_hop(d, s)
                pr[(s, 0, d)] = plane_hop(d, s, 0)
        for d in ("R", "L"):
            zd[(2, d)].wait_recv()
            pr[(3, 0, d)] = plane_hop(d, 3, 0)
        for h in (1, 2):
            for r in range(P_DIM):
                for d in ("R", "L"):
                    pr[(r, h - 1, d)].wait_recv()
                    pr[(r, h, d)] = plane_hop(d, r, h)
        for r in range(P_DIM):
            pr[(r, 2, "R")].wait_recv()
            pr[(r, 2, "L")].wait_recv()
        for desc in (list(zd.values()) + list(pr.values())
                     + list(prs.values()) + list(zrs.values())):
            desc.wait_send()

    return pl.pallas_call(
        body,
        out_shape=jax.ShapeDtypeStruct((m, n), bf16),
        in_specs=[
            pl.BlockSpec(memory_space=pltpu.VMEM),
            pl.BlockSpec(memory_space=pltpu.VMEM),
        ],
        out_specs=pl.BlockSpec(memory_space=pltpu.VMEM),
        scratch_shapes=[
            pltpu.VMEM((m, n), bf16),
            pltpu.VMEM((Q_DIM, qm, nh), bf16),
            pltpu.VMEM((Q_DIM, qm, nh), bf16),
            pltpu.VMEM((P_DIM, zm, nh), bf16),
            pltpu.VMEM((P_DIM, zm, nh), bf16),
            pltpu.SemaphoreType.DMA((P_DIM, Q_DIM - 1, 4)),
            pltpu.SemaphoreType.DMA((P_DIM - 1, 4)),
            pltpu.SemaphoreType.DMA((P_DIM - 1, 4)),
            pltpu.SemaphoreType.DMA((P_DIM, Q_DIM - 1, 4)),
        ],
        compiler_params=pltpu.CompilerParams(
            collective_id=0,
            vmem_limit_bytes=100 * 1024 * 1024,
        ),
    )(A, B)


# device time: 113463 ns/iter; 2.9886x vs baseline; 1.0007x over previous
import jax
import jax.numpy as jnp
from jax import lax
from jax.experimental import pallas as pl
from jax.experimental.pallas import tpu as pltpu

N_DEV = 16
P_DIM = 4
Q_DIM = 4


def kernel(A, B):
    m, k = A.shape
    k2, n = B.shape
    assert k == k2
    qm = m // Q_DIM
    zm = qm // P_DIM
    nh = n // 2

    f32 = jnp.float32
    bf16 = jnp.bfloat16

    def body(a_ref, b_ref, out_ref, acc_ref, pbufR, pbufL,
             zbufR, zbufL,
             pa_sems, zr_sems, za_sems, pg_sems):
        me = lax.axis_index("i")
        p = me // Q_DIM
        q = lax.rem(me, Q_DIM)
        rightq = p * Q_DIM + lax.rem(q + 1, Q_DIM)
        leftq = p * Q_DIM + lax.rem(q + (Q_DIM - 1), Q_DIM)
        zright = lax.rem(p + 1, P_DIM) * Q_DIM + q
        zleft = lax.rem(p + (P_DIM - 1), P_DIM) * Q_DIM + q

        barrier_sem = pltpu.get_barrier_semaphore()
        for nbr in (leftq, rightq, zleft, zright):
            pl.semaphore_signal(
                barrier_sem, inc=1,
                device_id=(nbr,), device_id_type=pl.DeviceIdType.MESH,
            )
        pl.semaphore_wait(barrier_sem, 4)

        def qrows(c):
            return pl.ds(c * qm, qm)

        def zrows(c):
            return pl.ds(c * zm, zm)

        def mm_quarter(c):
            acc_ref[qrows(c), :] = jnp.dot(
                a_ref[qrows(c), :].astype(bf16),
                b_ref[:, :].astype(bf16),
                preferred_element_type=f32,
            ).astype(bf16)

        oRq = lax.rem(q + 1, Q_DIM)
        oLq = lax.rem(q + (Q_DIM - 1), Q_DIM)
        ozR = lax.rem(p + 1, P_DIM)
        ozL = lax.rem(p + (P_DIM - 1), P_DIM)

        def tR(j):
            if j < P_DIM - 1:
                return lax.rem(p - j + P_DIM, P_DIM)
            return lax.rem(p + 1, P_DIM)

        def tL(j):
            if j < P_DIM - 1:
                return lax.rem(p + j, P_DIM)
            return lax.rem(p + (P_DIM - 1), P_DIM)

        def mm_slab(cq, cz):
            rows = pl.ds(cq * qm + cz * zm, zm)
            acc_ref[rows, :] = jnp.dot(
                a_ref[rows, :].astype(bf16),
                b_ref[:, :].astype(bf16),
                preferred_element_type=f32,
            ).astype(bf16)

        prs = {}

        def plane_rs_hop(d, j, h, src, t):
            if d == "R":
                src_ref = (acc_ref.at[pl.ds(q * qm + t * zm, zm), :nh]
                           if src is None else pbufR.at[src, zrows(t), :])
                desc = pltpu.make_async_remote_copy(
                    src_ref=src_ref,
                    dst_ref=pbufR.at[h, zrows(t), :],
                    send_sem=pa_sems.at[j, h, 0], recv_sem=pa_sems.at[j, h, 1],
                    device_id=(rightq,), device_id_type=pl.DeviceIdType.MESH,
                )
            else:
                src_ref = (acc_ref.at[pl.ds(q * qm + t * zm, zm), nh:]
                           if src is None else pbufL.at[src, zrows(t), :])
                desc = pltpu.make_async_remote_copy(
                    src_ref=src_ref,
                    dst_ref=pbufL.at[h, zrows(t), :],
                    send_sem=pa_sems.at[j, h, 2], recv_sem=pa_sems.at[j, h, 3],
                    device_id=(leftq,), device_id_type=pl.DeviceIdType.MESH,
                )
            desc.start()
            prs[(d, j, h)] = desc

        mm_slab(q, lax.rem(p, P_DIM))
        plane_rs_hop("R", 0, 0, None, tR(0))
        plane_rs_hop("L", 0, 0, None, tL(0))
        mm_slab(q, lax.rem(p + (P_DIM - 1), P_DIM))
        plane_rs_hop("R", 1, 0, None, tR(1))
        mm_slab(q, lax.rem(p + 1, P_DIM))
        plane_rs_hop("L", 1, 0, None, tL(1))
        mm_slab(q, lax.rem(p + 2, P_DIM))
        plane_rs_hop("R", 2, 0, None, tR(2))
        plane_rs_hop("L", 2, 0, None, tL(2))
        plane_rs_hop("R", 3, 0, None, tR(3))
        plane_rs_hop("L", 3, 0, None, tL(3))
        mm_quarter(lax.rem(q + 1, Q_DIM))
        mm_quarter(lax.rem(q + (Q_DIM - 1), Q_DIM))

        for h in (1, 2):
            qR_h = lax.rem(q - h + Q_DIM, Q_DIM)
            qL_h = lax.rem(q + h, Q_DIM)
            for j in range(P_DIM):
                t = tR(j)
                prs[("R", j, h - 1)].wait_recv()
                pbufR[h - 1, zrows(t), :] = (
                    pbufR[h - 1, zrows(t), :]
                    + acc_ref[pl.ds(qR_h * qm + t * zm, zm), :nh]
                )
                plane_rs_hop("R", j, h, h - 1, t)
                t = tL(j)
                prs[("L", j, h - 1)].wait_recv()
                pbufL[h - 1, zrows(t), :] = (
                    pbufL[h - 1, zrows(t), :]
                    + acc_ref[pl.ds(qL_h * qm + t * zm, zm), nh:]
                )
                plane_rs_hop("L", j, h, h - 1, t)
            if h == 1:
                mm_quarter(lax.rem(q + 2, Q_DIM))

        def plane_final_R(c):
            return (pbufR[Q_DIM - 2, zrows(c), :]
                    + acc_ref[pl.ds(oRq * qm + c * zm, zm), :nh])

        def plane_final_L(c):
            return (pbufL[Q_DIM - 2, zrows(c), :]
                    + acc_ref[pl.ds(oLq * qm + c * zm, zm), nh:])

        zrs = {}

        def z_rs_hop(d, s, src_slot):
            if d == "R":
                desc = pltpu.make_async_remote_copy(
                    src_ref=zbufR.at[src_slot], dst_ref=zbufR.at[s],
                    send_sem=zr_sems.at[s, 0], recv_sem=zr_sems.at[s, 1],
                    device_id=(zright,), device_id_type=pl.DeviceIdType.MESH,
                )
            else:
                desc = pltpu.make_async_remote_copy(
                    src_ref=zbufL.at[src_slot], dst_ref=zbufL.at[s],
                    send_sem=zr_sems.at[s, 2], recv_sem=zr_sems.at[s, 3],
                    device_id=(zleft,), device_id_type=pl.DeviceIdType.MESH,
                )
            desc.start()
            zrs[(d, s)] = desc

        prs[("R", 0, 2)].wait_recv()
        zbufR[P_DIM - 1, :, :] = plane_final_R(tR(0))
        z_rs_hop("R", 0, P_DIM - 1)
        prs[("L", 0, 2)].wait_recv()
        zbufL[P_DIM - 1, :, :] = plane_final_L(tL(0))
        z_rs_hop("L", 0, P_DIM - 1)
        for s in (1, 2):
            zrs[("R", s - 1)].wait_recv()
            prs[("R", s, 2)].wait_recv()
            zbufR[s - 1, :, :] = zbufR[s - 1, :, :] + plane_final_R(tR(s))
            z_rs_hop("R", s, s - 1)
            zrs[("L", s - 1)].wait_recv()
            prs[("L", s, 2)].wait_recv()
            zbufL[s - 1, :, :] = zbufL[s - 1, :, :] + plane_final_L(tL(s))
            z_rs_hop("L", s, s - 1)

        def gelu(z):
            return 0.5 * z * (
                1.0 + jnp.tanh(0.7978845608 * (z + 0.044715 * z * z * z))
            )

        def z_hop(d, s):
            if d == "R":
                g = lax.rem(p + 1 - s + P_DIM, P_DIM)
                rows = pl.ds(oRq * qm + g * zm, zm)
                desc = pltpu.make_async_remote_copy(
                    src_ref=out_ref.at[rows, :nh],
                    dst_ref=out_ref.at[rows, :nh],
                    send_sem=za_sems.at[s, 0], recv_sem=za_sems.at[s, 1],
                    device_id=(zright,), device_id_type=pl.DeviceIdType.MESH,
                )
            else:
                g = lax.rem(p - 1 + s + P_DIM, P_DIM)
                rows = pl.ds(oLq * qm + g * zm, zm)
                desc = pltpu.make_async_remote_copy(
                    src_ref=out_ref.at[rows, nh:],
                    dst_ref=out_ref.at[rows, nh:],
                    send_sem=za_sems.at[s, 2], recv_sem=za_sems.at[s, 3],
                    device_id=(zleft,), device_id_type=pl.DeviceIdType.MESH,
                )
            desc.start()
            return desc

        def plane_hop(d, r, h):
            if d == "R":
                cz = lax.rem(p + 1 - r + P_DIM, P_DIM)
                qh = lax.rem(q + 1 - h + Q_DIM, Q_DIM)
                rows = pl.ds(qh * qm + cz * zm, zm)
                desc = pltpu.make_async_remote_copy(
                    src_ref=out_ref.at[rows, :nh],
                    dst_ref=out_ref.at[rows, :nh],
                    send_sem=pg_sems.at[r, h, 0], recv_sem=pg_sems.at[r, h, 1],
                    device_id=(rightq,), device_id_type=pl.DeviceIdType.MESH,
                )
            else:
                cz = lax.rem(p - 1 + r + P_DIM, P_DIM)
                qh = lax.rem(q - 1 + h + Q_DIM, Q_DIM)
                rows = pl.ds(qh * qm + cz * zm, zm)
                desc = pltpu.make_async_remote_copy(
                    src_ref=out_ref.at[rows, nh:],
                    dst_ref=out_ref.at[rows, nh:],
                    send_sem=pg_sems.at[r, h, 2], recv_sem=pg_sems.at[r, h, 3],
                    device_id=(leftq,), device_id_type=pl.DeviceIdType.MESH,
                )
            desc.start()
            return desc

        zd = {}
        pr = {}
        zrs[("R", 2)].wait_recv()
        prs[("R", 3, 2)].wait_recv()
        zR = zbufR[P_DIM - 2, :, :].astype(f32) + plane_final_R(ozR)
        out_ref[pl.ds(oRq * qm + ozR * zm, zm), :nh] = gelu(zR).astype(bf16)
        zd[(0, "R")] = z_hop("R", 0)
        pr[(0, 0, "R")] = plane_hop("R", 0, 0)
        zrs[("L", 2)].wait_recv()
        prs[("L", 3, 2)].wait_recv()
        zL = zbufL[P_DIM - 2, :, :].astype(f32) + plane_final_L(ozL)
        out_ref[pl.ds(oLq * qm + ozL * zm, zm), nh:] = gelu(zL).astype(bf16)
        zd[(0, "L")] = z_hop("L", 0)
        pr[(0, 0, "L")] = plane_hop("L", 0, 0)
        for s in (1, 2):
            for d in ("R", "L"):
                zd[(s - 1, d)].wait_recv()
                zd[(s, d)] = z_hop(d, s)
                pr[(s, 0, d)] = plane_hop(d, s, 0)
        for d in ("R", "L"):
            zd[(2, d)].wait_recv()
            pr[(3, 0, d)] = plane_hop(d, 3, 0)
        for h in (1, 2):
            for r in range(P_DIM):
                for d in ("R", "L"):
                    pr[(r, h - 1, d)].wait_recv()
                    pr[(r, h, d)] = plane_hop(d, r, h)
        for r in range(P_DIM):
            pr[(r, 2, "R")].wait_recv()
            pr[(r, 2, "L")].wait_recv()
        for desc in (list(zd.values()) + list(pr.values())
                     + list(prs.values()) + list(zrs.values())):
            desc.wait_send()

    return pl.pallas_call(
        body,
        out_shape=jax.ShapeDtypeStruct((m, n), bf16),
        in_specs=[
            pl.BlockSpec(memory_space=pltpu.VMEM),
            pl.BlockSpec(memory_space=pltpu.VMEM),
        ],
        out_specs=pl.BlockSpec(memory_space=pltpu.VMEM),
        scratch_shapes=[
            pltpu.VMEM((m, n), bf16),
            pltpu.VMEM((Q_DIM, qm, nh), bf16),
            pltpu.VMEM((Q_DIM, qm, nh), bf16),
            pltpu.VMEM((P_DIM, zm, nh), bf16),
            pltpu.VMEM((P_DIM, zm, nh), bf16),
            pltpu.SemaphoreType.DMA((P_DIM, Q_DIM - 1, 4)),
            pltpu.SemaphoreType.DMA((P_DIM - 1, 4)),
            pltpu.SemaphoreType.DMA((P_DIM - 1, 4)),
            pltpu.SemaphoreType.DMA((P_DIM, Q_DIM - 1, 4)),
        ],
        compiler_params=pltpu.CompilerParams(
            collective_id=0,
            vmem_limit_bytes=100 * 1024 * 1024,
        ),
    )(A, B)


# device time: 110384 ns/iter; 3.0720x vs baseline; 1.0279x over previous
import jax
import jax.numpy as jnp
from jax import lax
from jax.experimental import pallas as pl
from jax.experimental.pallas import tpu as pltpu

N_DEV = 16
P_DIM = 4
Q_DIM = 4


def kernel(A, B):
    m, k = A.shape
    k2, n = B.shape
    assert k == k2
    qm = m // Q_DIM
    zm = qm // P_DIM
    nh = n // 2

    f32 = jnp.float32
    bf16 = jnp.bfloat16

    def body(a_ref, b_ref, out_ref, acc_ref, pbufR, pbufL,
             zbufR, zbufL,
             pa_sems, zr_sems, za_sems, pg_sems):
        me = lax.axis_index("i")
        p = me // Q_DIM
        q = lax.rem(me, Q_DIM)
        rightq = p * Q_DIM + lax.rem(q + 1, Q_DIM)
        leftq = p * Q_DIM + lax.rem(q + (Q_DIM - 1), Q_DIM)
        zright = lax.rem(p + 1, P_DIM) * Q_DIM + q
        zleft = lax.rem(p + (P_DIM - 1), P_DIM) * Q_DIM + q

        barrier_sem = pltpu.get_barrier_semaphore()
        for nbr in (leftq, rightq, zleft, zright):
            pl.semaphore_signal(
                barrier_sem, inc=1,
                device_id=(nbr,), device_id_type=pl.DeviceIdType.MESH,
            )
        pl.semaphore_wait(barrier_sem, 4)

        def qrows(c):
            return pl.ds(c * qm, qm)

        def zrows(c):
            return pl.ds(c * zm, zm)

        def mm_quarter(c):
            acc_ref[qrows(c), :] = jnp.dot(
                a_ref[qrows(c), :].astype(bf16),
                b_ref[:, :].astype(bf16),
                preferred_element_type=f32,
            ).astype(bf16)

        oRq = lax.rem(q + 1, Q_DIM)
        oLq = lax.rem(q + (Q_DIM - 1), Q_DIM)
        ozR = lax.rem(p + 1, P_DIM)
        ozL = lax.rem(p + (P_DIM - 1), P_DIM)

        def tR(j):
            if j < P_DIM - 1:
                return lax.rem(p - j + P_DIM, P_DIM)
            return lax.rem(p + 1, P_DIM)

        def tL(j):
            if j < P_DIM - 1:
                return lax.rem(p + j, P_DIM)
            return lax.rem(p + (P_DIM - 1), P_DIM)

        def mm_slab(cq, cz):
            rows = pl.ds(cq * qm + cz * zm, zm)
            acc_ref[rows, :] = jnp.dot(
                a_ref[rows, :].astype(bf16),
                b_ref[:, :].astype(bf16),
                preferred_element_type=f32,
            ).astype(bf16)

        prs = {}

        def plane_rs_hop(d, j, h, src, t):
            if d == "R":
                src_ref = (acc_ref.at[pl.ds(q * qm + t * zm, zm), :nh]
                           if src is None else pbufR.at[src, zrows(t), :])
                desc = pltpu.make_async_remote_copy(
                    src_ref=src_ref,
                    dst_ref=pbufR.at[h, zrows(t), :],
                    send_sem=pa_sems.at[j, h, 0], recv_sem=pa_sems.at[j, h, 1],
                    device_id=(rightq,), device_id_type=pl.DeviceIdType.MESH,
                )
            else:
                src_ref = (acc_ref.at[pl.ds(q * qm + t * zm, zm), nh:]
                           if src is None else pbufL.at[src, zrows(t), :])
                desc = pltpu.make_async_remote_copy(
                    src_ref=src_ref,
                    dst_ref=pbufL.at[h, zrows(t), :],
                    send_sem=pa_sems.at[j, h, 2], recv_sem=pa_sems.at[j, h, 3],
                    device_id=(leftq,), device_id_type=pl.DeviceIdType.MESH,
                )
            desc.start()
            prs[(d, j, h)] = desc

        mm_slab(q, lax.rem(p, P_DIM))
        plane_rs_hop("R", 0, 0, None, tR(0))
        plane_rs_hop("L", 0, 0, None, tL(0))
        mm_slab(q, lax.rem(p + (P_DIM - 1), P_DIM))
        plane_rs_hop("R", 1, 0, None, tR(1))
        mm_slab(q, lax.rem(p + 1, P_DIM))
        plane_rs_hop("L", 1, 0, None, tL(1))
        mm_slab(q, lax.rem(p + 2, P_DIM))
        plane_rs_hop("R", 2, 0, None, tR(2))
        plane_rs_hop("L", 2, 0, None, tL(2))
        plane_rs_hop("R", 3, 0, None, tR(3))
        plane_rs_hop("L", 3, 0, None, tL(3))
        mm_quarter(lax.rem(q + 1, Q_DIM))
        mm_quarter(lax.rem(q + (Q_DIM - 1), Q_DIM))

        def rs_step(d, j, h):
            if d == "R":
                t = tR(j)
                qh = lax.rem(q - h + Q_DIM, Q_DIM)
                prs[("R", j, h - 1)].wait_recv()
                pbufR[h - 1, zrows(t), :] = (
                    pbufR[h - 1, zrows(t), :]
                    + acc_ref[pl.ds(qh * qm + t * zm, zm), :nh]
                )
                plane_rs_hop("R", j, h, h - 1, t)
            else:
                t = tL(j)
                qh = lax.rem(q + h, Q_DIM)
                prs[("L", j, h - 1)].wait_recv()
                pbufL[h - 1, zrows(t), :] = (
                    pbufL[h - 1, zrows(t), :]
                    + acc_ref[pl.ds(qh * qm + t * zm, zm), nh:]
                )
                plane_rs_hop("L", j, h, h - 1, t)

        def plane_final_R(c):
            return (pbufR[Q_DIM - 2, zrows(c), :]
                    + acc_ref[pl.ds(oRq * qm + c * zm, zm), :nh])

        def plane_final_L(c):
            return (pbufL[Q_DIM - 2, zrows(c), :]
                    + acc_ref[pl.ds(oLq * qm + c * zm, zm), nh:])

        zrs = {}

        def z_rs_hop(d, s, src_slot):
            if d == "R":
                desc = pltpu.make_async_remote_copy(
                    src_ref=zbufR.at[src_slot], dst_ref=zbufR.at[s],
                    send_sem=zr_sems.at[s, 0], recv_sem=zr_sems.at[s, 1],
                    device_id=(zright,), device_id_type=pl.DeviceIdType.MESH,
                )
            else:
                desc = pltpu.make_async_remote_copy(
                    src_ref=zbufL.at[src_slot], dst_ref=zbufL.at[s],
                    send_sem=zr_sems.at[s, 2], recv_sem=zr_sems.at[s, 3],
                    device_id=(zleft,), device_id_type=pl.DeviceIdType.MESH,
                )
            desc.start()
            zrs[(d, s)] = desc

        def z_step(d, s):
            if s == 0:
                prs[(d, 0, 2)].wait_recv()
                if d == "R":
                    zbufR[P_DIM - 1, :, :] = plane_final_R(tR(0))
                else:
                    zbufL[P_DIM - 1, :, :] = plane_final_L(tL(0))
                z_rs_hop(d, 0, P_DIM - 1)
            else:
                zrs[(d, s - 1)].wait_recv()
                prs[(d, s, 2)].wait_recv()
                if d == "R":
                    zbufR[s - 1, :, :] = zbufR[s - 1, :, :] + plane_final_R(tR(s))
                else:
                    zbufL[s - 1, :, :] = zbufL[s - 1, :, :] + plane_final_L(tL(s))
                z_rs_hop(d, s, s - 1)

        q2 = lax.rem(q + 2, Q_DIM)
        rs_step("R", 0, 1)
        rs_step("L", 0, 1)
        rs_step("R", 1, 1)
        rs_step("L", 1, 1)
        mm_slab(q2, lax.rem(p, P_DIM))
        rs_step("R", 0, 2)
        rs_step("L", 0, 2)
        rs_step("R", 2, 1)
        rs_step("L", 2, 1)
        mm_slab(q2, lax.rem(p + (P_DIM - 1), P_DIM))
        mm_slab(q2, lax.rem(p + 1, P_DIM))
        rs_step("R", 1, 2)
        rs_step("L", 1, 2)
        rs_step("R", 3, 1)
        rs_step("L", 3, 1)
        mm_slab(q2, lax.rem(p + 2, P_DIM))
        z_step("R", 0)
        z_step("L", 0)
        rs_step("R", 2, 2)
        rs_step("L", 2, 2)
        z_step("R", 1)
        z_step("L", 1)
        rs_step("R", 3, 2)
        rs_step("L", 3, 2)
        z_step("R", 2)
        z_step("L", 2)

        def gelu(z):
            return 0.5 * z * (
                1.0 + jnp.tanh(0.7978845608 * (z + 0.044715 * z * z * z))
            )

        def z_hop(d, s):
            if d == "R":
                g = lax.rem(p + 1 - s + P_DIM, P_DIM)
                rows = pl.ds(oRq * qm + g * zm, zm)
                desc = pltpu.make_async_remote_copy(
                    src_ref=out_ref.at[rows, :nh],
                    dst_ref=out_ref.at[rows, :nh],
                    send_sem=za_sems.at[s, 0], recv_sem=za_sems.at[s, 1],
                    device_id=(zright,), device_id_type=pl.DeviceIdType.MESH,
                )
            else:
                g = lax.rem(p - 1 + s + P_DIM, P_DIM)
                rows = pl.ds(oLq * qm + g * zm, zm)
                desc = pltpu.make_async_remote_copy(
                    src_ref=out_ref.at[rows, nh:],
                    dst_ref=out_ref.at[rows, nh:],
                    send_sem=za_sems.at[s, 2], recv_sem=za_sems.at[s, 3],
                    device_id=(zleft,), device_id_type=pl.DeviceIdType.MESH,
                )
            desc.start()
            return desc

        def plane_hop(d, r, h):
            if d == "R":
                cz = lax.rem(p + 1 - r + P_DIM, P_DIM)
                qh = lax.rem(q + 1 - h + Q_DIM, Q_DIM)
                rows = pl.ds(qh * qm + cz * zm, zm)
                desc = pltpu.make_async_remote_copy(
                    src_ref=out_ref.at[rows, :nh],
                    dst_ref=out_ref.at[rows, :nh],
                    send_sem=pg_sems.at[r, h, 0], recv_sem=pg_sems.at[r, h, 1],
                    device_id=(rightq,), device_id_type=pl.DeviceIdType.MESH,
                )
            else:
                cz = lax.rem(p - 1 + r + P_DIM, P_DIM)
                qh = lax.rem(q - 1 + h + Q_DIM, Q_DIM)
                rows = pl.ds(qh * qm + cz * zm, zm)
                desc = pltpu.make_async_remote_copy(
                    src_ref=out_ref.at[rows, nh:],
                    dst_ref=out_ref.at[rows, nh:],
                    send_sem=pg_sems.at[r, h, 2], recv_sem=pg_sems.at[r, h, 3],
                    device_id=(leftq,), device_id_type=pl.DeviceIdType.MESH,
                )
            desc.start()
            return desc

        zd = {}
        pr = {}
        zrs[("R", 2)].wait_recv()
        prs[("R", 3, 2)].wait_recv()
        zR = zbufR[P_DIM - 2, :, :].astype(f32) + plane_final_R(ozR)
        out_ref[pl.ds(oRq * qm + ozR * zm, zm), :nh] = gelu(zR).astype(bf16)
        zrs[("L", 2)].wait_recv()
        prs[("L", 3, 2)].wait_recv()
        zL = zbufL[P_DIM - 2, :, :].astype(f32) + plane_final_L(ozL)
        out_ref[pl.ds(oLq * qm + ozL * zm, zm), nh:] = gelu(zL).astype(bf16)
        zd[(0, "R")] = z_hop("R", 0)
        pr[(0, 0, "R")] = plane_hop("R", 0, 0)
        zd[(0, "L")] = z_hop("L", 0)
        pr[(0, 0, "L")] = plane_hop("L", 0, 0)
        for s in (1, 2):
            for d in ("R", "L"):
                zd[(s - 1, d)].wait_recv()
                zd[(s, d)] = z_hop(d, s)
                pr[(s, 0, d)] = plane_hop(d, s, 0)
        for d in ("R", "L"):
            zd[(2, d)].wait_recv()
            pr[(3, 0, d)] = plane_hop(d, 3, 0)
        for h in (1, 2):
            for r in range(P_DIM):
                for d in ("R", "L"):
                    pr[(r, h - 1, d)].wait_recv()
                    pr[(r, h, d)] = plane_hop(d, r, h)
        for r in range(P_DIM):
            pr[(r, 2, "R")].wait_recv()
            pr[(r, 2, "L")].wait_recv()
        for desc in (list(zd.values()) + list(pr.values())
                     + list(prs.values()) + list(zrs.values())):
            desc.wait_send()

    return pl.pallas_call(
        body,
        out_shape=jax.ShapeDtypeStruct((m, n), bf16),
        in_specs=[
            pl.BlockSpec(memory_space=pltpu.VMEM),
            pl.BlockSpec(memory_space=pltpu.VMEM),
        ],
        out_specs=pl.BlockSpec(memory_space=pltpu.VMEM),
        scratch_shapes=[
            pltpu.VMEM((m, n), bf16),
            pltpu.VMEM((Q_DIM, qm, nh), bf16),
            pltpu.VMEM((Q_DIM, qm, nh), bf16),
            pltpu.VMEM((P_DIM, zm, nh), bf16),
            pltpu.VMEM((P_DIM, zm, nh), bf16),
            pltpu.SemaphoreType.DMA((P_DIM, Q_DIM - 1, 4)),
            pltpu.SemaphoreType.DMA((P_DIM - 1, 4)),
            pltpu.SemaphoreType.DMA((P_DIM - 1, 4)),
            pltpu.SemaphoreType.DMA((P_DIM, Q_DIM - 1, 4)),
        ],
        compiler_params=pltpu.CompilerParams(
            collective_id=0,
            vmem_limit_bytes=100 * 1024 * 1024,
        ),
    )(A, B)
